# baseline (speedup 1.0000x reference)
import functools

import jax
import jax.numpy as jnp
from jax import lax
from jax.experimental import pallas as pl
from jax.experimental.pallas import tpu as pltpu
from jax.experimental.pallas import tpu_sc as plsc

EMBED = 32
L = 16


@functools.lru_cache(maxsize=None)
def _make_sc_kernel(B: int, NC: int, NS: int):
    NW = NC * NS
    b_per_w = B // NW
    nchunks = b_per_w // L
    mesh = plsc.VectorSubcoreMesh(core_axis_name="c", subcore_axis_name="s")

    @functools.partial(
        pl.kernel,
        mesh=mesh,
        compiler_params=pltpu.CompilerParams(
            needs_layout_passes=False, use_tc_tiling_on_sc=False),
        out_type=jax.ShapeDtypeStruct((B,), jnp.float32),
        scratch_types=[
            pltpu.VMEM((b_per_w,), jnp.int32),
            pltpu.VMEM((b_per_w,), jnp.int32),
            pltpu.VMEM((EMBED, b_per_w), jnp.float32),
            pltpu.VMEM((EMBED, b_per_w), jnp.float32),
            pltpu.VMEM((b_per_w,), jnp.float32),
            pltpu.VMEM((b_per_w,), jnp.float32),
            pltpu.VMEM((b_per_w,), jnp.float32),
            pltpu.SemaphoreType.DMA,
        ],
    )
    def k(user_hbm, item_hbm, uet_hbm, iet_hbm, ubt_hbm, ibt_hbm, out_hbm,
          uidx, iidx, uev, iev, ubv, ibv, outv, sem):
        wid = lax.axis_index("s") * NC + lax.axis_index("c")
        base = wid * b_per_w

        pltpu.sync_copy(user_hbm.at[pl.ds(base, b_per_w)], uidx)
        pltpu.sync_copy(item_hbm.at[pl.ds(base, b_per_w)], iidx)

        def adjust(c, carry):
            sl = pl.ds(c * L, L)
            uidx[sl] = jnp.maximum(uidx[sl] - 1, 0)
            iidx[sl] = jnp.maximum(iidx[sl] - 1, 0)
            return carry

        lax.fori_loop(0, nchunks, adjust, 0)

        copies = []
        copies.append(pltpu.async_copy(ubt_hbm.at[0].at[uidx], ubv, sem))
        copies.append(pltpu.async_copy(ibt_hbm.at[0].at[iidx], ibv, sem))
        for c in range(EMBED):
            copies.append(
                pltpu.async_copy(uet_hbm.at[c].at[uidx], uev.at[c], sem))
            copies.append(
                pltpu.async_copy(iet_hbm.at[c].at[iidx], iev.at[c], sem))
        for cp in copies:
            cp.wait()

        def chunk(c, carry):
            sl = pl.ds(c * L, L)
            acc = ubv[sl] + ibv[sl]
            for d in range(EMBED):
                acc = acc + uev[d, sl] * iev[d, sl]
            outv[sl] = acc
            return carry

        lax.fori_loop(0, nchunks, chunk, 0)

        pltpu.sync_copy(outv, out_hbm.at[pl.ds(base, b_per_w)])

    return k


def kernel(user, item, user_e, item_e, user_b, item_b):
    B = user.shape[0]
    info = plsc.get_sparse_core_info()
    k = _make_sc_kernel(B, info.num_cores, info.num_subcores)
    return k(user.astype(jnp.int32), item.astype(jnp.int32),
             user_e.T, item_e.T, user_b.T, item_b.T)


# speedup vs baseline: 5.9399x; 5.9399x over previous
import functools

import jax
import jax.numpy as jnp
from jax import lax
from jax.experimental import pallas as pl
from jax.experimental.pallas import tpu as pltpu
from jax.experimental.pallas import tpu_sc as plsc

EMBED = 32
L = 16


@functools.lru_cache(maxsize=None)
def _make_sc_kernel(B: int, NC: int, NS: int):
    NW = NC * NS
    b_per_w = B // NW
    nchunks = b_per_w // L
    mesh = plsc.VectorSubcoreMesh(core_axis_name="c", subcore_axis_name="s")

    @functools.partial(
        pl.kernel,
        mesh=mesh,
        compiler_params=pltpu.CompilerParams(
            needs_layout_passes=False, use_tc_tiling_on_sc=False),
        out_type=jax.ShapeDtypeStruct((B,), jnp.float32),
        scratch_types=[
            pltpu.VMEM((b_per_w,), jnp.int32),
            pltpu.VMEM((b_per_w,), jnp.int32),
            pltpu.VMEM((b_per_w, EMBED), jnp.float32),
            pltpu.VMEM((b_per_w, EMBED), jnp.float32),
            pltpu.VMEM((b_per_w,), jnp.float32),
            pltpu.VMEM((b_per_w,), jnp.float32),
            pltpu.VMEM((b_per_w,), jnp.float32),
            pltpu.SemaphoreType.DMA,
            pltpu.SemaphoreType.DMA,
            pltpu.SemaphoreType.DMA,
            pltpu.SemaphoreType.DMA,
        ],
    )
    def k(user_hbm, item_hbm, ue_hbm, ie_hbm, ubt_hbm, ibt_hbm, out_hbm,
          uidx, iidx, ue_rows, ie_rows, ubv, ibv, outv, s0, s1, s2, s3):
        wid = lax.axis_index("s") * NC + lax.axis_index("c")
        base = wid * b_per_w

        pltpu.sync_copy(user_hbm.at[pl.ds(base, b_per_w)], uidx)
        pltpu.sync_copy(item_hbm.at[pl.ds(base, b_per_w)], iidx)

        def adjust(c, carry):
            sl = pl.ds(c * L, L)
            uidx[sl] = jnp.maximum(uidx[sl] - 1, 0)
            iidx[sl] = jnp.maximum(iidx[sl] - 1, 0)
            return carry

        lax.fori_loop(0, nchunks, adjust, 0)

        c0 = pltpu.async_copy(ue_hbm.at[uidx], ue_rows, s0)
        c1 = pltpu.async_copy(ie_hbm.at[iidx], ie_rows, s1)
        c2 = pltpu.async_copy(ubt_hbm.at[0].at[uidx], ubv, s2)
        c3 = pltpu.async_copy(ibt_hbm.at[0].at[iidx], ibv, s3)
        c0.wait()
        c1.wait()
        c2.wait()
        c3.wait()

        iota = lax.iota(jnp.int32, L)

        def chunk(c, carry):
            sl = pl.ds(c * L, L)
            acc = ubv[sl] + ibv[sl]
            row = c * L + iota
            for d in range(EMBED):
                col = jnp.full((L,), d, jnp.int32)
                acc = acc + (plsc.load_gather(ue_rows, [row, col])
                             * plsc.load_gather(ie_rows, [row, col]))
            outv[sl] = acc
            return carry

        lax.fori_loop(0, nchunks, chunk, 0)

        pltpu.sync_copy(outv, out_hbm.at[pl.ds(base, b_per_w)])

    return k


def kernel(user, item, user_e, item_e, user_b, item_b):
    B = user.shape[0]
    info = plsc.get_sparse_core_info()
    k = _make_sc_kernel(B, info.num_cores, info.num_subcores)
    return k(user.astype(jnp.int32), item.astype(jnp.int32),
             user_e, item_e, user_b.T, item_b.T)
